# baseline (speedup 1.0000x reference)
import functools

import jax
import jax.numpy as jnp
from jax import lax
from jax.experimental import pallas as pl
from jax.experimental.pallas import tpu as pltpu
from jax.experimental.pallas import tpu_sc as plsc

N_TOK = 65536
C = 1024
D = 64

BN = 2048
NB = N_TOK // BN

NC = 2
NS = 16
NW = NC * NS
B_PER_W = N_TOK // NW


def _codebook_kernel(mp_ref, eps_ref, phi_ref, phit2_ref, p2_ref):
    mp = mp_ref[...]
    eps = eps_ref[...]
    phi_ref[...] = jnp.dot(mp, eps, preferred_element_type=jnp.float32)
    phit = lax.dot_general(eps, mp, (((0,), (1,)), ((), ())),
                           preferred_element_type=jnp.float32)
    phit2_ref[...] = -2.0 * phit
    p2_ref[...] = jnp.sum(phit * phit, axis=0, keepdims=True)


def _assign_kernel(x_ref, phit2_ref, p2_ref, idx_ref):
    x = x_ref[...]
    xp2 = jnp.dot(x, phit2_ref[...], preferred_element_type=jnp.float32)
    d2 = xp2 + p2_ref[...]
    idx = jnp.argmin(d2, axis=1).astype(jnp.int32)
    idx_ref[0, 0, :] = idx


ROUND_ROWS = 256
N_ROUNDS = B_PER_W // ROUND_ROWS


def _gather_body(phi_hbm, idx_hbm, out_hbm, phi_v, idx_v, out_v, sem, osem):
    wid = lax.axis_index("s") * NC + lax.axis_index("c")
    pltpu.sync_copy(phi_hbm, phi_v)
    pltpu.sync_copy(idx_hbm.at[wid], idx_v)
    base = wid * B_PER_W

    def do_round(r, buf):
        def group(g, _):
            cvec = idx_v[pl.ds(r * ROUND_ROWS + g * 16, 16)]
            for jb in range(4):
                cs = [cvec[4 * jb + t] for t in range(4)]
                vals = [phi_v[cs[t], pl.ds(w * 16, 16)]
                        for t in range(4) for w in range(D // 16)]
                for t in range(4):
                    for w in range(D // 16):
                        out_v[buf, g * 16 + 4 * jb + t, pl.ds(w * 16, 16)] = (
                            vals[4 * t + w])
            return 0

        lax.fori_loop(0, ROUND_ROWS // 16, group, 0)
        return pltpu.async_copy(
            out_v.at[buf], out_hbm.at[pl.ds(base + r * ROUND_ROWS, ROUND_ROWS)],
            osem)

    pending = do_round(0, 0)
    for r in range(1, N_ROUNDS):
        nxt = do_round(r, r % 2)
        pending.wait()
        pending = nxt
    pending.wait()


@functools.cache
def _sc_gather():
    return pl.kernel(
        _gather_body,
        out_type=jax.ShapeDtypeStruct((N_TOK, D), jnp.float32),
        mesh=plsc.VectorSubcoreMesh(core_axis_name="c", subcore_axis_name="s"),
        scratch_types=[
            pltpu.VMEM((C, D), jnp.float32),
            pltpu.VMEM((B_PER_W,), jnp.int32),
            pltpu.VMEM((2, ROUND_ROWS, D), jnp.float32),
            pltpu.SemaphoreType.DMA,
            pltpu.SemaphoreType.DMA,
        ],
        compiler_params=pltpu.CompilerParams(use_tc_tiling_on_sc=False),
    )


def kernel(X, matrix_parents, epsilon):
    phi, phit, p2 = pl.pallas_call(
        _codebook_kernel,
        out_shape=[
            jax.ShapeDtypeStruct((C, D), jnp.float32),
            jax.ShapeDtypeStruct((D, C), jnp.float32),
            jax.ShapeDtypeStruct((1, C), jnp.float32),
        ],
    )(matrix_parents, epsilon)

    idx2 = pl.pallas_call(
        _assign_kernel,
        grid=(NB,),
        in_specs=[
            pl.BlockSpec((BN, D), lambda i: (i, 0)),
            pl.BlockSpec((D, C), lambda i: (0, 0)),
            pl.BlockSpec((1, C), lambda i: (0, 0)),
        ],
        out_specs=pl.BlockSpec((1, 1, BN), lambda i: (i, 0, 0)),
        out_shape=jax.ShapeDtypeStruct((NB, 1, BN), jnp.int32),
        compiler_params=pltpu.CompilerParams(
            dimension_semantics=("arbitrary",)),
    )(X, phit, p2)

    idx = idx2.reshape(NW, B_PER_W)
    return _sc_gather()(phi, idx)


# speedup vs baseline: 1.0012x; 1.0012x over previous
import functools

import jax
import jax.numpy as jnp
from jax import lax
from jax.experimental import pallas as pl
from jax.experimental.pallas import tpu as pltpu
from jax.experimental.pallas import tpu_sc as plsc

N_TOK = 65536
C = 1024
D = 64

BN = 1024
NB = N_TOK // BN

NC = 2
NS = 16
NW = NC * NS
B_PER_W = N_TOK // NW


def _codebook_kernel(mp_ref, eps_ref, phi_ref, phit2_ref, p2_ref):
    mp = mp_ref[...]
    eps = eps_ref[...]
    phi_ref[...] = jnp.dot(mp, eps, preferred_element_type=jnp.float32)
    phit = lax.dot_general(eps, mp, (((0,), (1,)), ((), ())),
                           preferred_element_type=jnp.float32)
    phit2_ref[...] = -2.0 * phit
    p2_ref[...] = jnp.sum(phit * phit, axis=0, keepdims=True)


def _assign_kernel(x_ref, phit2_ref, p2_ref, idx_ref):
    x = x_ref[...]
    xp2 = jnp.dot(x, phit2_ref[...], preferred_element_type=jnp.float32)
    d2 = xp2 + p2_ref[...]
    idx = jnp.argmin(d2, axis=1, keepdims=True).astype(jnp.int32)
    idx_ref[...] = idx


ROUND_ROWS = 256
N_ROUNDS = B_PER_W // ROUND_ROWS


def _gather_body(phi_hbm, idx_hbm, out_hbm, phi_v, idx_v, out_v, sem, osem):
    wid = lax.axis_index("s") * NC + lax.axis_index("c")
    pltpu.sync_copy(phi_hbm, phi_v)
    pltpu.sync_copy(idx_hbm.at[pl.ds(wid * B_PER_W, B_PER_W)], idx_v)
    base = wid * B_PER_W

    def do_round(r, buf):
        def group(g, _):
            cvec = jnp.reshape(
                idx_v[pl.ds(r * ROUND_ROWS + g * 16, 16), :], (16,))
            for jb in range(4):
                cs = [cvec[4 * jb + t] for t in range(4)]
                vals = [phi_v[cs[t], pl.ds(w * 16, 16)]
                        for t in range(4) for w in range(D // 16)]
                for t in range(4):
                    for w in range(D // 16):
                        out_v[buf, g * 16 + 4 * jb + t, pl.ds(w * 16, 16)] = (
                            vals[4 * t + w])
            return 0

        lax.fori_loop(0, ROUND_ROWS // 16, group, 0)
        return pltpu.async_copy(
            out_v.at[buf], out_hbm.at[pl.ds(base + r * ROUND_ROWS, ROUND_ROWS)],
            osem)

    pending = do_round(0, 0)
    for r in range(1, N_ROUNDS):
        nxt = do_round(r, r % 2)
        pending.wait()
        pending = nxt
    pending.wait()


@functools.cache
def _sc_gather():
    return pl.kernel(
        _gather_body,
        out_type=jax.ShapeDtypeStruct((N_TOK, D), jnp.float32),
        mesh=plsc.VectorSubcoreMesh(core_axis_name="c", subcore_axis_name="s"),
        scratch_types=[
            pltpu.VMEM((C, D), jnp.float32),
            pltpu.VMEM((B_PER_W, 1), jnp.int32),
            pltpu.VMEM((2, ROUND_ROWS, D), jnp.float32),
            pltpu.SemaphoreType.DMA,
            pltpu.SemaphoreType.DMA,
        ],
        compiler_params=pltpu.CompilerParams(use_tc_tiling_on_sc=False),
    )


def kernel(X, matrix_parents, epsilon):
    phi, phit, p2 = pl.pallas_call(
        _codebook_kernel,
        out_shape=[
            jax.ShapeDtypeStruct((C, D), jnp.float32),
            jax.ShapeDtypeStruct((D, C), jnp.float32),
            jax.ShapeDtypeStruct((1, C), jnp.float32),
        ],
    )(matrix_parents, epsilon)

    idx2 = pl.pallas_call(
        _assign_kernel,
        grid=(NB,),
        in_specs=[
            pl.BlockSpec((BN, D), lambda i: (i, 0)),
            pl.BlockSpec((D, C), lambda i: (0, 0)),
            pl.BlockSpec((1, C), lambda i: (0, 0)),
        ],
        out_specs=pl.BlockSpec((BN, 1), lambda i: (i, 0)),
        out_shape=jax.ShapeDtypeStruct((N_TOK, 1), jnp.int32),
        compiler_params=pltpu.CompilerParams(
            dimension_semantics=("arbitrary",)),
    )(X, phit, p2)

    return _sc_gather()(phi, idx2)


# speedup vs baseline: 1.1743x; 1.1729x over previous
import jax
import jax.numpy as jnp
from jax import lax
from jax.experimental import pallas as pl
from jax.experimental.pallas import tpu as pltpu

N_TOK = 65536
C = 1024
D = 64
BN = 1024
NB = N_TOK // BN


def _codebook_kernel(mp_ref, eps_ref, phi_ref, phit2_ref, p2_ref):
    mp = mp_ref[...]
    eps = eps_ref[...]
    phi_ref[...] = jnp.dot(mp, eps, preferred_element_type=jnp.float32)
    phit = lax.dot_general(eps, mp, (((0,), (1,)), ((), ())),
                           preferred_element_type=jnp.float32)
    phit2_ref[...] = -2.0 * phit
    p2_ref[...] = jnp.sum(phit * phit, axis=0, keepdims=True)


def _vq_kernel(x_ref, phit2_ref, p2_ref, phi_ref, out_ref):
    x = x_ref[...]
    xp2 = jnp.dot(x, phit2_ref[...], preferred_element_type=jnp.float32)
    d2 = xp2 + p2_ref[...]
    idx = jnp.argmin(d2, axis=1, keepdims=True).astype(jnp.int32)
    ids = lax.broadcasted_iota(jnp.int32, d2.shape, 1)
    onehot = jnp.where(ids == idx, 1.0, 0.0)
    out_ref[...] = jnp.dot(onehot, phi_ref[...],
                           preferred_element_type=jnp.float32)


def kernel(X, matrix_parents, epsilon):
    phi, phit2, p2 = pl.pallas_call(
        _codebook_kernel,
        out_shape=[
            jax.ShapeDtypeStruct((C, D), jnp.float32),
            jax.ShapeDtypeStruct((D, C), jnp.float32),
            jax.ShapeDtypeStruct((1, C), jnp.float32),
        ],
    )(matrix_parents, epsilon)

    return pl.pallas_call(
        _vq_kernel,
        grid=(NB,),
        in_specs=[
            pl.BlockSpec((BN, D), lambda i: (i, 0)),
            pl.BlockSpec((D, C), lambda i: (0, 0)),
            pl.BlockSpec((1, C), lambda i: (0, 0)),
            pl.BlockSpec((C, D), lambda i: (0, 0)),
        ],
        out_specs=pl.BlockSpec((BN, D), lambda i: (i, 0)),
        out_shape=jax.ShapeDtypeStruct((N_TOK, D), jnp.float32),
        compiler_params=pltpu.CompilerParams(
            dimension_semantics=("arbitrary",)),
    )(X, phit2, p2, phi)
